# baseline (speedup 1.0000x reference)
import jax
import jax.numpy as jnp
from jax import lax
from jax.experimental import pallas as pl
from jax.experimental.pallas import tpu as pltpu
from jax.experimental.pallas import tpu_sc as plsc

_N = 10000
_E = 320000
_NG = 64
_NC = 2
_NS = 16
_NW = _NC * _NS
_EPT = _E // _NW
_CH = 80
_NCH = _EPT // _CH
_NPAD = 10240
_RPT = _NPAD // _NS
_DPT = _NPAD // _NS
_HW = 128
_SEG = 2000
_CPS = _SEG // _CH


def _sc_edge_pass(col_split):
  mesh = plsc.VectorSubcoreMesh(
      core_axis_name="c", subcore_axis_name="s", num_cores=_NC,
      num_subcores=_NS)

  ept = _E // _NS if col_split else _E // _NW
  nch = ept // _CH
  grp = 4

  def kbody(h_hbm, as_hbm, ad_hbm, src_hbm, dst_hbm, acc_hbm, den_hbm,
            as_t, ad_t, ex80_t, rows_t, rows64_t,
            zden_t, src80_t, dst80_t, stile_t, smod_t, acc_s, den_s, sem):
    cid = lax.axis_index("c")
    sid = lax.axis_index("s")
    widx = sid if col_split else cid * _NS + sid
    coff = cid * 64 if col_split else 0

    pltpu.sync_copy(as_hbm, as_t)
    pltpu.sync_copy(ad_hbm, ad_t)

    zf = jnp.zeros((16,), jnp.float32)

    def zdfill(i, _):
      zden_t[pl.ds(i * 16, 16)] = zf
      return 0
    lax.fori_loop(0, _DPT // 16, zdfill, 0)

    def zacc(k, _):
      pltpu.sync_copy(zden_t,
                      acc_s.at[pl.ds(sid * _RPT * 64 + k * _DPT, _DPT)])
      return 0
    lax.fori_loop(0, _RPT * 64 // _DPT, zacc, 0)
    pltpu.sync_copy(zden_t, den_s.at[pl.ds(sid * _DPT, _DPT)])
    plsc.subcore_barrier()

    def chunk(c, _):
      base = widx * ept + c * 16
      pltpu.sync_copy(src_hbm.at[pl.ds(base, 16)], src80_t)
      pltpu.sync_copy(dst_hbm.at[pl.ds(base, 16)], dst80_t)

      sv = src80_t[...]
      dv = dst80_t[...]
      av = plsc.load_gather(as_t, [sv])
      bv = plsc.load_gather(ad_t, [dv])
      v = av + bv
      exv = jnp.exp(jnp.maximum(v, 0.2 * v))
      ex80_t[...] = exv
      sb = sv * _HW + coff
      db = dv * 64

      for cc in range(64):
        stile_t[...] = sb + cc
        pltpu.async_copy(h_hbm.at[stile_t], rows_t, sem).wait()
        smod_t[...] = db + cc
        rows64_t[...] = rows_t[...] * exv
        pltpu.sync_copy(rows64_t, acc_s.at[smod_t], add=True)

      pltpu.sync_copy(ex80_t, den_s.at[dst80_t], add=True)
      return 0
    lax.fori_loop(0, ept // 16, chunk, 0)

    plsc.subcore_barrier()

    base = sid * _RPT * 64
    pltpu.sync_copy(acc_s.at[pl.ds(base, _RPT * 64)],
                    acc_hbm.at[cid, pl.ds(base, _RPT * 64)])
    dbase = sid * _DPT
    pltpu.sync_copy(den_s.at[pl.ds(dbase, _DPT)],
                    den_hbm.at[cid, pl.ds(dbase, _DPT)])

  return pl.kernel(
      kbody,
      out_type=(jax.ShapeDtypeStruct((_NC, _NPAD * 64), jnp.float32),
                jax.ShapeDtypeStruct((_NC, _NPAD), jnp.float32)),
      mesh=mesh,
      scratch_types=(
          pltpu.VMEM((_N,), jnp.float32),
          pltpu.VMEM((_N,), jnp.float32),
          pltpu.VMEM((16,), jnp.float32),
          pltpu.VMEM((16,), jnp.float32),
          pltpu.VMEM((16,), jnp.float32),
          pltpu.VMEM((_DPT,), jnp.float32),
          pltpu.VMEM((16,), jnp.int32),
          pltpu.VMEM((16,), jnp.int32),
          pltpu.VMEM((16,), jnp.int32),
          pltpu.VMEM((16,), jnp.int32),
          pltpu.VMEM_SHARED((_NPAD * 64,), jnp.float32),
          pltpu.VMEM_SHARED((_NPAD,), jnp.float32),
          pltpu.SemaphoreType.DMA,
      ),
      compiler_params=pltpu.CompilerParams(needs_layout_passes=False),
      name=f"gat_sc_edge_{'col' if col_split else 'part'}",
  )


def _tc1_body(x_ref, w1_ref, acat_ref, h_ref, aux_ref):
  h = jnp.dot(x_ref[...], w1_ref[...], preferred_element_type=jnp.float32)
  al = jnp.dot(h, acat_ref[...], preferred_element_type=jnp.float32)
  a_s = al[:, 0:1]
  a_d = al[:, 1:2]
  v = a_s + a_d
  selfex = jnp.exp(jnp.maximum(v, 0.2 * v))
  h_ref[...] = jnp.concatenate([h, jnp.zeros_like(h)], axis=1)
  aux_ref[...] = jnp.concatenate([a_s, a_d, selfex, jnp.zeros_like(selfex)],
                                 axis=1)


def _tc_mid_body(h_ref, acc_ref, den_ref, aux_ref, b_ref, w2_ref, acat_ref,
                 h2_ref, aux2_ref):
  h = h_ref[:, :64]
  selfex = aux_ref[:, 2:3]
  den = den_ref[0, :_N] + den_ref[1, :_N] + selfex + 1e-16
  num = acc_ref[0, :_N] + acc_ref[1, :_N] + selfex * h
  hm = jnp.maximum(num / den + b_ref[...], 0.0)
  h2 = jnp.dot(hm, w2_ref[...], preferred_element_type=jnp.float32)
  al = jnp.dot(h2, acat_ref[...], preferred_element_type=jnp.float32)
  a_s = al[:, 0:1]
  a_d = al[:, 1:2]
  v = a_s + a_d
  selfex2 = jnp.exp(jnp.maximum(v, 0.2 * v))
  h2_ref[...] = h2
  aux2_ref[...] = jnp.concatenate([a_s, a_d, selfex2, jnp.zeros_like(selfex2)],
                                  axis=1)


def _tc3_body(h_ref, acc_ref, den_ref, aux_ref, b_ref, batch_ref, wfc_ref,
              bfc_ref, out_ref):
  h = h_ref[...]
  selfex = aux_ref[:, 2:3]
  den = den_ref[0, :_N] + selfex + 1e-16
  num = (jnp.concatenate([acc_ref[0, :_N], acc_ref[1, :_N]], axis=1)
         + selfex * h)
  hfin = jnp.maximum(num / den + b_ref[...], 0.0)
  gid = lax.broadcasted_iota(jnp.int32, (_NG, _N), 0)
  mask = (batch_ref[...] == gid).astype(jnp.float32)
  sums = jnp.dot(mask, hfin, preferred_element_type=jnp.float32)
  cnt = jnp.sum(mask, axis=1, keepdims=True)
  pooled = sums / jnp.maximum(cnt, 1.0)
  out_ref[...] = jnp.dot(pooled, wfc_ref[...],
                         preferred_element_type=jnp.float32) + bfc_ref[...]


def kernel(x, edge_index, batch, W1, a1_src, a1_dst, b1, W2, a2_src, a2_dst,
           b2, W_fc, b_fc):
  f32 = jnp.float32
  hid1 = W1.shape[1]
  hid2 = W2.shape[1]

  srcf = edge_index[0]
  dstf = edge_index[1]
  acat1 = jnp.stack([a1_src, a1_dst], axis=1)
  acat2 = jnp.stack([a2_src, a2_dst], axis=1)
  b1r = b1.reshape(1, hid1)
  b2r = b2.reshape(1, hid2)
  bfcr = b_fc.reshape(1, -1)
  batch2 = batch.reshape(1, _N)

  h1p, aux1 = pl.pallas_call(
      _tc1_body,
      out_shape=(jax.ShapeDtypeStruct((_N, _HW), f32),
                 jax.ShapeDtypeStruct((_N, 4), f32)),
  )(x, W1, acat1)

  acc1, den1 = _sc_edge_pass(False)(h1p.reshape(_N * _HW),
                                    aux1[:, 0], aux1[:, 1], srcf, dstf)
  acc1 = acc1.reshape(_NC, _NPAD, 64)

  h2, aux2 = pl.pallas_call(
      _tc_mid_body,
      out_shape=(jax.ShapeDtypeStruct((_N, hid2), f32),
                 jax.ShapeDtypeStruct((_N, 4), f32)),
  )(h1p, acc1, den1.reshape(_NC, _NPAD, 1), aux1, b1r, W2, acat2)

  acc2, den2 = _sc_edge_pass(True)(h2.reshape(_N * _HW),
                                   aux2[:, 0], aux2[:, 1], srcf, dstf)
  acc2 = acc2.reshape(_NC, _NPAD, 64)

  out = pl.pallas_call(
      _tc3_body,
      out_shape=jax.ShapeDtypeStruct((_NG, W_fc.shape[1]), f32),
  )(h2, acc2, den2.reshape(_NC, _NPAD, 1), aux2, b2r, batch2, W_fc, bfcr)
  return out
